# baseline (speedup 1.0000x reference)
import functools

import jax
import jax.numpy as jnp
from jax import lax
from jax.experimental import pallas as pl
from jax.experimental.pallas import tpu as pltpu
from jax.experimental.pallas import tpu_sc as plsc

_NA = 50000
_NB = 50000
_NANG = 100000
_ND = 150000
_NP = 1600000

_C = 400
_G = _C // 16
_NW = 32
_EPS0 = 1e-12


def _rsqrt(x):
    i = plsc.bitcast(x, jnp.int32)
    i = 0x5F3759DF - (i >> 1)
    y = plsc.bitcast(i, jnp.float32)
    for _ in range(3):
        y = y * (1.5 - 0.5 * x * y * y)
    return y


def _sqrt(x):
    return x * _rsqrt(x)


def _acos(x):
    a = jnp.abs(x)
    u = jnp.maximum(1.0 - a, _EPS0)
    s = _sqrt(u)
    p = 1.5707288 + a * (-0.2121144 + a * (0.0742610 - 0.0187293 * a))
    r = s * p
    return jnp.where(x < 0.0, jnp.float32(3.14159265) - r, r)


def _col(ref, r16, c):
    return plsc.load_gather(ref, [r16, jnp.full((16,), c, jnp.int32)])


def _body(pos3, packed6, bi0, bi1, bk, br, ai0, ai1, ai2, ak, at,
          di0, di1, di2, di3, pi0, pi1, p_msk, out,
          ic, ics, r4, r6, co, mskb, acc, sg0, sg1, sg2):
    cid = lax.axis_index("c")
    sid = lax.axis_index("s")
    wid = sid * 2 + cid
    iota = lax.iota(jnp.int32, 16)
    acc[...] = jnp.zeros((16,), jnp.float32)
    sem_g = (sg0, sg1, sg2)

    def accumulate(e):
        acc[...] = acc[...] + e

    def pipelined(nch, prefetch, fire, finish):
        cnt = (nch - wid + _NW - 1) // _NW
        niter = (cnt + 3) // 3

        @pl.when(cnt > 0)
        def _prologue():
            prefetch(wid, 0)

        def body(j, carry):
            for u in range(3):
                k3 = 3 * j + u
                ck = wid + k3 * _NW

                @pl.when(k3 < cnt)
                def _f():
                    fire(ck, u)

                @pl.when(k3 + 1 < cnt)
                def _i():
                    prefetch(ck + _NW, (u + 1) % 3)

                @pl.when((k3 >= 1) & (k3 <= cnt))
                def _c():
                    finish(ck - _NW, (u + 2) % 3)

            return carry

        lax.fori_loop(0, niter, body, 0)

    def make_term(icols, ccols, msk, tbl, rows, compute):
        n = len(icols)

        def prefetch(c, b):
            sl = pl.ds(c * _C, _C)
            for s, col in enumerate(icols):
                pltpu.sync_copy(col.at[sl], ics.at[b, s])
            for s, col in enumerate(ccols):
                pltpu.sync_copy(col.at[sl], co.at[b, s])
            if msk is not None:
                pltpu.sync_copy(msk.at[sl], mskb.at[b])

        def fire(c, b):
            def rep(g, carry):
                o = pl.ds(g * 16, 16)
                for s in range(n):
                    ic[b, s, o] = ics[b, s, o]
                return carry

            lax.fori_loop(0, _G, rep, 0)
            for s in range(n):
                pltpu.async_copy(tbl.at[ic.at[b, s]], rows.at[b, s],
                                 sem_g[b])

        def finish(c, b):
            for s in range(n):
                pltpu.make_async_copy(
                    tbl.at[ic.at[b, s]], rows.at[b, s], sem_g[b]).wait()
            compute(c, b)

        return prefetch, fire, finish

    def bond_compute(c, b):
        def grp(g, carry):
            r16 = g * 16 + iota
            o = pl.ds(g * 16, 16)
            dx = _col(r4.at[b, 0], r16, 0) - _col(r4.at[b, 1], r16, 0)
            dy = _col(r4.at[b, 0], r16, 1) - _col(r4.at[b, 1], r16, 1)
            dz = _col(r4.at[b, 0], r16, 2) - _col(r4.at[b, 1], r16, 2)
            d2 = dx * dx + dy * dy + dz * dz + _EPS0
            d = _sqrt(d2)
            dd = d - co[b, 1, o]
            accumulate(co[b, 0, o] * dd * dd)
            return carry

        lax.fori_loop(0, _G, grp, 0)

    pipelined(_NB // _C, *make_term((bi0, bi1), (bk, br), None, pos3, r4,
                                    bond_compute))

    def angle_compute(c, b):
        def grp(g, carry):
            r16 = g * 16 + iota
            o = pl.ds(g * 16, 16)
            x2 = _col(r4.at[b, 1], r16, 0)
            y2 = _col(r4.at[b, 1], r16, 1)
            z2 = _col(r4.at[b, 1], r16, 2)
            v1x = _col(r4.at[b, 0], r16, 0) - x2
            v1y = _col(r4.at[b, 0], r16, 1) - y2
            v1z = _col(r4.at[b, 0], r16, 2) - z2
            v2x = _col(r4.at[b, 2], r16, 0) - x2
            v2y = _col(r4.at[b, 2], r16, 1) - y2
            v2z = _col(r4.at[b, 2], r16, 2) - z2
            n1sq = v1x * v1x + v1y * v1y + v1z * v1z + _EPS0
            n2sq = v2x * v2x + v2y * v2y + v2z * v2z + _EPS0
            dot = v1x * v2x + v1y * v2y + v1z * v2z
            cos_t = jnp.clip(dot * _rsqrt(n1sq * n2sq), -0.999999, 0.999999)
            dt = _acos(cos_t) - co[b, 1, o]
            accumulate(co[b, 0, o] * dt * dt)
            return carry

        lax.fori_loop(0, _G, grp, 0)

    pipelined(_NANG // _C, *make_term((ai0, ai1, ai2), (ak, at), None,
                                      pos3, r4, angle_compute))

    def dih_compute(c, b):
        def grp(g, carry):
            r16 = g * 16 + iota
            p1x = _col(r4.at[b, 0], r16, 0)
            p1y = _col(r4.at[b, 0], r16, 1)
            p1z = _col(r4.at[b, 0], r16, 2)
            p2x = _col(r4.at[b, 1], r16, 0)
            p2y = _col(r4.at[b, 1], r16, 1)
            p2z = _col(r4.at[b, 1], r16, 2)
            p3x = _col(r4.at[b, 2], r16, 0)
            p3y = _col(r4.at[b, 2], r16, 1)
            p3z = _col(r4.at[b, 2], r16, 2)
            b1x = p2x - p1x
            b1y = p2y - p1y
            b1z = p2z - p1z
            b2x = p3x - p2x
            b2y = p3y - p2y
            b2z = p3z - p2z
            b3x = _col(r4.at[b, 3], r16, 0) - p3x
            b3y = _col(r4.at[b, 3], r16, 1) - p3y
            b3z = _col(r4.at[b, 3], r16, 2) - p3z
            c1x = b1y * b2z - b1z * b2y
            c1y = b1z * b2x - b1x * b2z
            c1z = b1x * b2y - b1y * b2x
            c2x = b2y * b3z - b2z * b3y
            c2y = b2z * b3x - b2x * b3z
            c2z = b2x * b3y - b2y * b3x
            n1sq = c1x * c1x + c1y * c1y + c1z * c1z + _EPS0
            n2sq = c2x * c2x + c2y * c2y + c2z * c2z + _EPS0
            dot = c1x * c2x + c1y * c2y + c1z * c2z
            cos_p = jnp.clip(dot * _rsqrt(n1sq * n2sq), -0.999999, 0.999999)
            accumulate(1.0 + cos_p)
            return carry

        lax.fori_loop(0, _G, grp, 0)

    pipelined(_ND // _C, *make_term((di0, di1, di2, di3), (), None, pos3,
                                    r4, dih_compute))

    def pair_compute(c, b):
        def grp(g, carry):
            r16 = g * 16 + iota
            dx = _col(r6.at[b, 0], r16, 0) - _col(r6.at[b, 1], r16, 0)
            dy = _col(r6.at[b, 0], r16, 1) - _col(r6.at[b, 1], r16, 1)
            dz = _col(r6.at[b, 0], r16, 2) - _col(r6.at[b, 1], r16, 2)
            r2 = dx * dx + dy * dy + dz * dz + 1.0
            inv_r = _rsqrt(r2)
            qq = _col(r6.at[b, 0], r16, 3) * _col(r6.at[b, 1], r16, 3)
            eps_ij = _col(r6.at[b, 0], r16, 4) * _col(r6.at[b, 1], r16, 4)
            sig_ij = 0.5 * (_col(r6.at[b, 0], r16, 5)
                            + _col(r6.at[b, 1], r16, 5))
            sr = sig_ij * inv_r
            sr2 = sr * sr
            sr6 = sr2 * sr2 * sr2
            e = 4.0 * eps_ij * (sr6 * sr6 - sr6) + 332.33 * qq * inv_r
            accumulate(mskb[b, pl.ds(g * 16, 16)] * e)
            return carry

        lax.fori_loop(0, _G, grp, 0)

    pipelined(_NP // _C, *make_term((pi0, pi1), (), p_msk, packed6, r6,
                                    pair_compute))

    pltpu.sync_copy(acc, out.at[wid])


@functools.partial(
    pl.kernel,
    out_type=jax.ShapeDtypeStruct((_NW, 16), jnp.float32),
    mesh=plsc.VectorSubcoreMesh(
        core_axis_name="c", subcore_axis_name="s", num_cores=2,
        num_subcores=16),
    compiler_params=pltpu.CompilerParams(
        needs_layout_passes=False, use_tc_tiling_on_sc=False),
    scratch_types=[
        pltpu.VMEM((3, 4, _C), jnp.int32),
        pltpu.VMEM((3, 4, _C), jnp.int32),
        pltpu.VMEM((3, 4, _C, 3), jnp.float32),
        pltpu.VMEM((3, 2, _C, 6), jnp.float32),
        pltpu.VMEM((3, 2, _C), jnp.float32),
        pltpu.VMEM((3, _C), jnp.float32),
        pltpu.VMEM((16,), jnp.float32),
        pltpu.SemaphoreType.DMA,
        pltpu.SemaphoreType.DMA,
        pltpu.SemaphoreType.DMA,
    ],
)
def _energy_sc(*args):
    _body(*args)


def kernel(atom_pos, sb_mask_e, charges, epsilon, sigma, bond_coeffs,
           angle_coeffs, bond_idx, angle_idx, dihedral_idx, pair_idx):
    packed6 = jnp.concatenate(
        [atom_pos, charges[:, None], jnp.sqrt(epsilon)[:, None],
         sigma[:, None]], axis=1)
    bond_idx = bond_idx.astype(jnp.int32)
    angle_idx = angle_idx.astype(jnp.int32)
    dihedral_idx = dihedral_idx.astype(jnp.int32)
    pair_idx = pair_idx.astype(jnp.int32)
    partials = _energy_sc(
        atom_pos, packed6,
        bond_idx[:, 0], bond_idx[:, 1],
        bond_coeffs[:, 0], bond_coeffs[:, 1],
        angle_idx[:, 0], angle_idx[:, 1], angle_idx[:, 2],
        angle_coeffs[:, 0], angle_coeffs[:, 1],
        dihedral_idx[:, 0], dihedral_idx[:, 1], dihedral_idx[:, 2],
        dihedral_idx[:, 3],
        pair_idx[:, 0], pair_idx[:, 1], sb_mask_e)
    return jnp.sum(partials)
